# baseline (speedup 1.0000x reference)
import jax
import jax.numpy as jnp
from jax.experimental import pallas as pl
from jax.experimental.pallas import tpu as pltpu


def _fused_softmax_matmul(sel_ref, items_ref, out_ref):
    x = sel_ref[...]
    m = jnp.max(x, axis=-1, keepdims=True)
    e = jnp.exp(x - m)
    s = jnp.sum(e, axis=-1, keepdims=True)
    acc = jnp.dot(e, items_ref[...], preferred_element_type=jnp.float32)
    out_ref[...] = acc / s


def kernel(selections, items):
    batch, n_items = selections.shape
    n_items2, n_samples = items.shape
    assert n_items == n_items2
    tile_b = 512
    grid = (batch // tile_b,)
    return pl.pallas_call(
        _fused_softmax_matmul,
        grid=grid,
        in_specs=[
            pl.BlockSpec((tile_b, n_items), lambda i: (i, 0)),
            pl.BlockSpec((n_items, n_samples), lambda i: (0, 0)),
        ],
        out_specs=pl.BlockSpec((tile_b, n_samples), lambda i: (i, 0)),
        out_shape=jax.ShapeDtypeStruct((batch, n_samples), jnp.float32),
        compiler_params=pltpu.CompilerParams(
            dimension_semantics=("parallel",),
        ),
    )(selections, items)


# speedup vs baseline: 1.1398x; 1.1398x over previous
import jax
import jax.numpy as jnp
from jax.experimental import pallas as pl
from jax.experimental.pallas import tpu as pltpu


def _fused_softmax_matmul(sel_ref, items_ref, out_ref):
    x = sel_ref[...]
    m = jnp.max(x, axis=-1, keepdims=True)
    e = jnp.exp(x - m)
    s = jnp.sum(e, axis=-1, keepdims=True)
    acc = jnp.dot(e, items_ref[...], preferred_element_type=jnp.float32)
    out_ref[...] = acc / s


def kernel(selections, items):
    batch, n_items = selections.shape
    n_items2, n_samples = items.shape
    assert n_items == n_items2
    tile_b = 2048
    grid = (batch // tile_b,)
    return pl.pallas_call(
        _fused_softmax_matmul,
        grid=grid,
        in_specs=[
            pl.BlockSpec((tile_b, n_items), lambda i: (i, 0)),
            pl.BlockSpec((n_items, n_samples), lambda i: (0, 0)),
        ],
        out_specs=pl.BlockSpec((tile_b, n_samples), lambda i: (i, 0)),
        out_shape=jax.ShapeDtypeStruct((batch, n_samples), jnp.float32),
        compiler_params=pltpu.CompilerParams(
            dimension_semantics=("parallel",),
        ),
    )(selections, items)


# speedup vs baseline: 1.1475x; 1.0067x over previous
import jax
import jax.numpy as jnp
from jax.experimental import pallas as pl
from jax.experimental.pallas import tpu as pltpu


def _fused_softmax_matmul(sel_ref, items_ref, out_ref):
    x = sel_ref[...]
    m = jnp.max(x, axis=-1, keepdims=True)
    e = jnp.exp(x - m)
    s = jnp.sum(e, axis=-1, keepdims=True)
    acc = jnp.dot(e, items_ref[...], preferred_element_type=jnp.float32)
    out_ref[...] = acc / s


def kernel(selections, items):
    batch, n_items = selections.shape
    n_items2, n_samples = items.shape
    assert n_items == n_items2
    tile_b = 4096
    grid = (batch // tile_b,)
    return pl.pallas_call(
        _fused_softmax_matmul,
        grid=grid,
        in_specs=[
            pl.BlockSpec((tile_b, n_items), lambda i: (i, 0)),
            pl.BlockSpec((n_items, n_samples), lambda i: (0, 0)),
        ],
        out_specs=pl.BlockSpec((tile_b, n_samples), lambda i: (i, 0)),
        out_shape=jax.ShapeDtypeStruct((batch, n_samples), jnp.float32),
        compiler_params=pltpu.CompilerParams(
            dimension_semantics=("parallel",),
        ),
    )(selections, items)
